# baseline (speedup 1.0000x reference)
import jax
import jax.numpy as jnp
from jax.experimental import pallas as pl
from jax.experimental.pallas import tpu as pltpu
from jax.experimental.pallas import tpu_sc as plsc

DIM = 1024
HEADS = 16
HEAD_DIM = 64
NEXP = 64
NTOK = 2048
SCALE = HEAD_DIM ** (-0.5)
SWITCHLOSS = 0.1
ZLOSS = 0.001
TILE = 256
NT = NTOK // TILE


def _gating_body(x_ref, wg_ref, wkv_ref, bkv_ref,
                 kv_ref, g_ref, qidx_ref, par_ref, inv_ref, aux_ref,
                 imp_ref, frq_ref, zs_ref):
    i = pl.program_id(0)
    x = x_ref[...]
    logits = jnp.dot(x, wg_ref[...], preferred_element_type=jnp.float32)
    kv_ref[...] = (jnp.dot(x, wkv_ref[...], preferred_element_type=jnp.float32)
                   + bkv_ref[...])
    m = jnp.max(logits, axis=-1, keepdims=True)
    ex = jnp.exp(logits - m)
    denom = jnp.sum(ex, axis=-1, keepdims=True)
    probs = ex / denom
    lse = m + jnp.log(denom)
    eidx = jax.lax.broadcasted_iota(jnp.int32, (TILE, NEXP), 1)
    tglob = i * TILE + jax.lax.broadcasted_iota(jnp.int32, (TILE, 1), 0)
    cur = probs
    selmask = jnp.zeros((TILE, NEXP), jnp.float32)
    inv = jnp.zeros((TILE, NEXP), jnp.int32)
    gsum = jnp.zeros((TILE, 1), jnp.float32)
    gvals = []
    qidxs = []
    pars = []
    for k in range(HEADS):
        mk = jnp.max(cur, axis=-1, keepdims=True)
        ism = cur == mk
        first = jnp.min(jnp.where(ism, eidx, NEXP), axis=-1, keepdims=True)
        sel = eidx == first
        selmask = selmask + sel.astype(jnp.float32)
        inv = inv + jnp.where(sel, k * NTOK + tglob + 1, 0)
        gsum = gsum + mk
        gvals.append(mk)
        qidxs.append(tglob * (NEXP // 2) + jax.lax.shift_right_logical(first, 1))
        pars.append(jnp.bitwise_and(first, 1))
        cur = jnp.where(sel, -jnp.inf, cur)
    g_ref[...] = jnp.concatenate(gvals, axis=1) / (gsum + 1e-6)
    qidx_ref[...] = jnp.concatenate(qidxs, axis=1)
    par_ref[...] = jnp.concatenate(pars, axis=1)
    inv_ref[...] = inv

    @pl.when(i == 0)
    def _():
        imp_ref[...] = jnp.zeros_like(imp_ref)
        frq_ref[...] = jnp.zeros_like(frq_ref)
        zs_ref[...] = jnp.zeros_like(zs_ref)
        aux_ref[...] = jnp.zeros_like(aux_ref)

    imp_ref[...] += jnp.sum(probs, axis=0, keepdims=True)
    frq_ref[...] += jnp.sum(selmask, axis=0, keepdims=True)
    zs_ref[...] += jnp.sum(lse * lse).reshape(1, 1)

    @pl.when(i == NT - 1)
    def _():
        imp = imp_ref[...]
        frq = frq_ref[...]
        impn = imp / jnp.maximum(jnp.sum(jnp.abs(imp)), 1e-12)
        frqn = frq / jnp.maximum(jnp.sum(jnp.abs(frq)), 1e-12)
        switch = jnp.sum(impn * frqn) * NEXP
        zmean = zs_ref[0, 0] / NTOK
        aux_ref[...] = (SWITCHLOSS * switch + ZLOSS * zmean).reshape(1, 1)


def _allq_body(x_ref, wm_ref, out_ref):
    out_ref[...] = jnp.dot(x_ref[...], wm_ref[...],
                           preferred_element_type=jnp.float32)


def _attn_body(q_ref, k_ref, v_ref, g_ref, par_ref, y_ref):
    k = pl.program_id(1)
    kiota = jax.lax.broadcasted_iota(jnp.int32, (TILE, HEADS), 1)
    khot = kiota == k
    par = jnp.sum(jnp.where(khot, par_ref[...], 0), axis=1, keepdims=True)
    q128 = q_ref[0]
    q = jnp.where(par == 0, q128[:, :HEAD_DIM],
                  q128[:, HEAD_DIM:]).astype(jnp.bfloat16)
    s = jax.lax.dot_general(q, k_ref[...], (((1,), (1,)), ((), ())),
                            preferred_element_type=jnp.float32) * SCALE
    m = jnp.max(s, axis=-1, keepdims=True)
    p = jnp.exp(s - m)
    l = jnp.sum(p, axis=-1, keepdims=True)
    o = jnp.dot(p.astype(jnp.bfloat16), v_ref[...],
                preferred_element_type=jnp.float32)
    g = jnp.sum(jnp.where(khot, g_ref[...], 0.0), axis=1, keepdims=True)
    y_ref[0] = jnp.concatenate(
        [o * (g / l), jnp.zeros((TILE, HEAD_DIM), jnp.float32)], axis=1)


def _combine_body(z_ref, wo_ref, out_ref):
    out_ref[...] = jnp.dot(z_ref[...].astype(jnp.bfloat16), wo_ref[...],
                           preferred_element_type=jnp.float32)


def _sc_gather_rows(table, idx2d):
    nidx = idx2d.shape[1]
    width = table.shape[1]
    win = 128
    mesh = plsc.VectorSubcoreMesh(core_axis_name="core",
                                  subcore_axis_name="subcore")

    @pl.kernel(out_type=jax.ShapeDtypeStruct((nidx, width), table.dtype),
               mesh=mesh)
    def run(x_hbm, i_hbm, o_hbm):
        def body(i_vmem, o_vmem):
            pltpu.sync_copy(x_hbm.at[i_vmem.at[0]], o_vmem)

        pltpu.emit_pipeline(
            body,
            grid=(nidx // win,),
            in_specs=[pl.BlockSpec((1, win), index_map=lambda i: (0, i))],
            out_specs=[pl.BlockSpec((win, width), index_map=lambda i: (i, 0))],
            core_axis_name=("core", "subcore"),
            dimension_semantics=(pltpu.PARALLEL,),
        )(i_hbm, o_hbm)

    return run(table, idx2d)


def kernel(x, task_bh, W_gate, W_map, W_out, W_kv, b_kv):
    Bb, Nn, C = x.shape
    xf = x.reshape(NTOK, DIM)
    wg = jax.lax.dynamic_index_in_dim(W_gate, task_bh, axis=0, keepdims=False)

    kv, g16, qidx, par16, inv, aux = pl.pallas_call(
        _gating_body,
        grid=(NT,),
        in_specs=[
            pl.BlockSpec((TILE, DIM), lambda i: (i, 0)),
            pl.BlockSpec((DIM, NEXP), lambda i: (0, 0)),
            pl.BlockSpec((DIM, 2 * HEAD_DIM), lambda i: (0, 0)),
            pl.BlockSpec((1, 2 * HEAD_DIM), lambda i: (0, 0)),
        ],
        out_specs=[
            pl.BlockSpec((TILE, 2 * HEAD_DIM), lambda i: (i, 0)),
            pl.BlockSpec((TILE, HEADS), lambda i: (i, 0)),
            pl.BlockSpec((TILE, HEADS), lambda i: (i, 0)),
            pl.BlockSpec((TILE, HEADS), lambda i: (i, 0)),
            pl.BlockSpec((TILE, NEXP), lambda i: (i, 0)),
            pl.BlockSpec((1, 1), lambda i: (0, 0)),
        ],
        out_shape=[
            jax.ShapeDtypeStruct((NTOK, 2 * HEAD_DIM), jnp.float32),
            jax.ShapeDtypeStruct((NTOK, HEADS), jnp.float32),
            jax.ShapeDtypeStruct((NTOK, HEADS), jnp.int32),
            jax.ShapeDtypeStruct((NTOK, HEADS), jnp.int32),
            jax.ShapeDtypeStruct((NTOK, NEXP), jnp.int32),
            jax.ShapeDtypeStruct((1, 1), jnp.float32),
        ],
        scratch_shapes=[
            pltpu.VMEM((1, NEXP), jnp.float32),
            pltpu.VMEM((1, NEXP), jnp.float32),
            pltpu.VMEM((1, 1), jnp.float32),
        ],
    )(xf, wg, W_kv, b_kv.reshape(1, 2 * HEAD_DIM))

    xb = xf.astype(jnp.bfloat16)
    wm_flat = W_map.transpose(1, 0, 2).reshape(DIM, NEXP * HEAD_DIM)
    wm_flat = wm_flat.astype(jnp.bfloat16)
    allq = pl.pallas_call(
        _allq_body,
        grid=(NT,),
        in_specs=[
            pl.BlockSpec((TILE, DIM), lambda i: (i, 0)),
            pl.BlockSpec((DIM, NEXP * HEAD_DIM), lambda i: (0, 0)),
        ],
        out_specs=pl.BlockSpec((TILE, NEXP * HEAD_DIM), lambda i: (i, 0)),
        out_shape=jax.ShapeDtypeStruct((NTOK, NEXP * HEAD_DIM), jnp.float32),
    )(xb, wm_flat)

    qidx_hm = qidx.T.reshape(1, NTOK * HEADS)
    qrows = _sc_gather_rows(allq.reshape(NTOK * NEXP // 2, 2 * HEAD_DIM),
                            qidx_hm)
    qhm = qrows.reshape(HEADS, NTOK, 2 * HEAD_DIM)

    k_ = kv[:, :HEAD_DIM].astype(jnp.bfloat16)
    v_ = kv[:, HEAD_DIM:].astype(jnp.bfloat16)
    y = pl.pallas_call(
        _attn_body,
        grid=(NT, HEADS),
        in_specs=[
            pl.BlockSpec((1, TILE, 2 * HEAD_DIM), lambda i, k: (k, i, 0)),
            pl.BlockSpec((NTOK, HEAD_DIM), lambda i, k: (0, 0)),
            pl.BlockSpec((NTOK, HEAD_DIM), lambda i, k: (0, 0)),
            pl.BlockSpec((TILE, HEADS), lambda i, k: (i, 0)),
            pl.BlockSpec((TILE, HEADS), lambda i, k: (i, 0)),
        ],
        out_specs=pl.BlockSpec((1, TILE, 2 * HEAD_DIM), lambda i, k: (k, i, 0)),
        out_shape=jax.ShapeDtypeStruct((HEADS, NTOK, 2 * HEAD_DIM),
                                       jnp.float32),
    )(qhm, k_, v_, g16, par16)

    src = jnp.concatenate(
        [jnp.zeros((1, 2 * HEAD_DIM), jnp.float32),
         y.reshape(HEADS * NTOK, 2 * HEAD_DIM)], axis=0)
    zrows = _sc_gather_rows(src, inv.reshape(1, NTOK * NEXP))
    z = zrows.reshape(NTOK, NEXP * 2 * HEAD_DIM)

    wo_pad = jnp.pad(W_out, ((0, 0), (0, HEAD_DIM), (0, 0)))
    wo_pad = wo_pad.reshape(NEXP * 2 * HEAD_DIM, DIM).astype(jnp.bfloat16)
    out = pl.pallas_call(
        _combine_body,
        grid=(NT,),
        in_specs=[
            pl.BlockSpec((TILE, NEXP * 2 * HEAD_DIM), lambda i: (i, 0)),
            pl.BlockSpec((NEXP * 2 * HEAD_DIM, DIM), lambda i: (0, 0)),
        ],
        out_specs=pl.BlockSpec((TILE, DIM), lambda i: (i, 0)),
        out_shape=jax.ShapeDtypeStruct((NTOK, DIM), jnp.float32),
    )(z, wo_pad)

    return out.reshape(Bb, Nn, C), aux[0, 0]


# speedup vs baseline: 8.7150x; 8.7150x over previous
import jax
import jax.numpy as jnp
from jax.experimental import pallas as pl
from jax.experimental.pallas import tpu as pltpu
from jax.experimental.pallas import tpu_sc as plsc

DIM = 1024
HEADS = 16
HEAD_DIM = 64
NEXP = 64
NTOK = 2048
SCALE = HEAD_DIM ** (-0.5)
SWITCHLOSS = 0.1
ZLOSS = 0.001
TILE = 256
NT = NTOK // TILE


def _gating_body(x_ref, wg_ref, wkv_ref, bkv_ref,
                 kv_ref, g_ref, qidx_ref, par_ref, scat_ref, msk_ref, aux_ref,
                 imp_ref, frq_ref, zs_ref):
    i = pl.program_id(0)
    x = x_ref[...]
    logits = jnp.dot(x, wg_ref[...], preferred_element_type=jnp.float32)
    kv_ref[...] = (jnp.dot(x, wkv_ref[...], preferred_element_type=jnp.float32)
                   + bkv_ref[...])
    m = jnp.max(logits, axis=-1, keepdims=True)
    ex = jnp.exp(logits - m)
    denom = jnp.sum(ex, axis=-1, keepdims=True)
    probs = ex / denom
    lse = m + jnp.log(denom)
    eidx = jax.lax.broadcasted_iota(jnp.int32, (TILE, NEXP), 1)
    tglob = i * TILE + jax.lax.broadcasted_iota(jnp.int32, (TILE, 1), 0)
    cur = probs
    selmask = jnp.zeros((TILE, NEXP), jnp.float32)
    gsum = jnp.zeros((TILE, 1), jnp.float32)
    gvals = []
    qidxs = []
    pars = []
    scats = []
    for k in range(HEADS):
        mk = jnp.max(cur, axis=-1, keepdims=True)
        ism = cur == mk
        first = jnp.min(jnp.where(ism, eidx, NEXP), axis=-1, keepdims=True)
        sel = eidx == first
        selmask = selmask + sel.astype(jnp.float32)
        gsum = gsum + mk
        gvals.append(mk)
        qidxs.append(tglob * (NEXP // 2) + jax.lax.shift_right_logical(first, 1))
        pars.append(jnp.bitwise_and(first, 1))
        scats.append(tglob * NEXP + first)
        cur = jnp.where(sel, -jnp.inf, cur)
    g_ref[...] = jnp.concatenate(gvals, axis=1) / (gsum + 1e-6)
    qidx_ref[...] = jnp.concatenate(qidxs, axis=1)
    par_ref[...] = jnp.concatenate(pars, axis=1)
    scat_ref[...] = jnp.concatenate(scats, axis=1)
    msk_ref[...] = selmask

    @pl.when(i == 0)
    def _():
        imp_ref[...] = jnp.zeros_like(imp_ref)
        frq_ref[...] = jnp.zeros_like(frq_ref)
        zs_ref[...] = jnp.zeros_like(zs_ref)
        aux_ref[...] = jnp.zeros_like(aux_ref)

    imp_ref[...] += jnp.sum(probs, axis=0, keepdims=True)
    frq_ref[...] += jnp.sum(selmask, axis=0, keepdims=True)
    zs_ref[...] += jnp.sum(lse * lse).reshape(1, 1)

    @pl.when(i == NT - 1)
    def _():
        imp = imp_ref[...]
        frq = frq_ref[...]
        impn = imp / jnp.maximum(jnp.sum(jnp.abs(imp)), 1e-12)
        frqn = frq / jnp.maximum(jnp.sum(jnp.abs(frq)), 1e-12)
        switch = jnp.sum(impn * frqn) * NEXP
        zmean = zs_ref[0, 0] / NTOK
        aux_ref[...] = (SWITCHLOSS * switch + ZLOSS * zmean).reshape(1, 1)


def _allq_body(x_ref, wm_ref, out_ref):
    out_ref[...] = jnp.dot(x_ref[...], wm_ref[...],
                           preferred_element_type=jnp.float32)


def _attn_body(q_ref, k_ref, v_ref, g_ref, par_ref, y_ref):
    k = pl.program_id(1)
    kiota = jax.lax.broadcasted_iota(jnp.int32, (TILE, HEADS), 1)
    khot = kiota == k
    par = jnp.sum(jnp.where(khot, par_ref[...], 0), axis=1, keepdims=True)
    q128 = q_ref[0]
    q = jnp.where(par == 0, q128[:, :HEAD_DIM],
                  q128[:, HEAD_DIM:]).astype(jnp.bfloat16)
    s = jax.lax.dot_general(q, k_ref[...], (((1,), (1,)), ((), ())),
                            preferred_element_type=jnp.float32) * SCALE
    m = jnp.max(s, axis=-1, keepdims=True)
    p = jnp.exp(s - m)
    l = jnp.sum(p, axis=-1, keepdims=True)
    o = jnp.dot(p.astype(jnp.bfloat16), v_ref[...],
                preferred_element_type=jnp.float32)
    g = jnp.sum(jnp.where(khot, g_ref[...], 0.0), axis=1, keepdims=True)
    y_ref[0] = jnp.concatenate(
        [o * (g / l), jnp.zeros((TILE, HEAD_DIM), jnp.float32)], axis=1)


def _combine_body(z_ref, msk_ref, p_ref, wo_ref, out_ref):
    mask = jnp.dot(msk_ref[...].astype(jnp.bfloat16), p_ref[...],
                   preferred_element_type=jnp.float32)
    z = jnp.where(mask > 0.5, z_ref[...], 0.0)
    out_ref[...] = jnp.dot(z.astype(jnp.bfloat16), wo_ref[...],
                           preferred_element_type=jnp.float32)


def _sc_gather_rows(table, idx2d):
    nidx = idx2d.shape[1]
    width = table.shape[1]
    win = 128
    mesh = plsc.VectorSubcoreMesh(core_axis_name="core",
                                  subcore_axis_name="subcore")

    @pl.kernel(out_type=jax.ShapeDtypeStruct((nidx, width), table.dtype),
               mesh=mesh)
    def run(x_hbm, i_hbm, o_hbm):
        def body(i_vmem, o_vmem):
            pltpu.sync_copy(x_hbm.at[i_vmem.at[0]], o_vmem)

        pltpu.emit_pipeline(
            body,
            grid=(nidx // win,),
            in_specs=[pl.BlockSpec((1, win), index_map=lambda i: (0, i))],
            out_specs=[pl.BlockSpec((win, width), index_map=lambda i: (i, 0))],
            core_axis_name=("core", "subcore"),
            dimension_semantics=(pltpu.PARALLEL,),
        )(i_hbm, o_hbm)

    return run(table, idx2d)


def _sc_scatter_rows(rows, idx2d, nout):
    nidx = idx2d.shape[1]
    width = rows.shape[1]
    win = 128
    mesh = plsc.VectorSubcoreMesh(core_axis_name="core",
                                  subcore_axis_name="subcore")

    @pl.kernel(out_type=jax.ShapeDtypeStruct((nout, width), rows.dtype),
               mesh=mesh)
    def run(x_hbm, i_hbm, o_hbm):
        def body(x_vmem, i_vmem):
            pltpu.sync_copy(x_vmem, o_hbm.at[i_vmem.at[0]])

        pltpu.emit_pipeline(
            body,
            grid=(nidx // win,),
            in_specs=[pl.BlockSpec((win, width), index_map=lambda i: (i, 0)),
                      pl.BlockSpec((1, win), index_map=lambda i: (0, i))],
            out_specs=[],
            core_axis_name=("core", "subcore"),
            dimension_semantics=(pltpu.PARALLEL,),
        )(x_hbm, i_hbm)

    return run(rows, idx2d)


def kernel(x, task_bh, W_gate, W_map, W_out, W_kv, b_kv):
    Bb, Nn, C = x.shape
    xf = x.reshape(NTOK, DIM)
    wg = jax.lax.dynamic_index_in_dim(W_gate, task_bh, axis=0, keepdims=False)

    kv, g16, qidx, par16, scat, selmask, aux = pl.pallas_call(
        _gating_body,
        grid=(NT,),
        in_specs=[
            pl.BlockSpec((TILE, DIM), lambda i: (i, 0)),
            pl.BlockSpec((DIM, NEXP), lambda i: (0, 0)),
            pl.BlockSpec((DIM, 2 * HEAD_DIM), lambda i: (0, 0)),
            pl.BlockSpec((1, 2 * HEAD_DIM), lambda i: (0, 0)),
        ],
        out_specs=[
            pl.BlockSpec((TILE, 2 * HEAD_DIM), lambda i: (i, 0)),
            pl.BlockSpec((TILE, HEADS), lambda i: (i, 0)),
            pl.BlockSpec((TILE, HEADS), lambda i: (i, 0)),
            pl.BlockSpec((TILE, HEADS), lambda i: (i, 0)),
            pl.BlockSpec((TILE, HEADS), lambda i: (i, 0)),
            pl.BlockSpec((TILE, NEXP), lambda i: (i, 0)),
            pl.BlockSpec((1, 1), lambda i: (0, 0)),
        ],
        out_shape=[
            jax.ShapeDtypeStruct((NTOK, 2 * HEAD_DIM), jnp.float32),
            jax.ShapeDtypeStruct((NTOK, HEADS), jnp.float32),
            jax.ShapeDtypeStruct((NTOK, HEADS), jnp.int32),
            jax.ShapeDtypeStruct((NTOK, HEADS), jnp.int32),
            jax.ShapeDtypeStruct((NTOK, HEADS), jnp.int32),
            jax.ShapeDtypeStruct((NTOK, NEXP), jnp.float32),
            jax.ShapeDtypeStruct((1, 1), jnp.float32),
        ],
        scratch_shapes=[
            pltpu.VMEM((1, NEXP), jnp.float32),
            pltpu.VMEM((1, NEXP), jnp.float32),
            pltpu.VMEM((1, 1), jnp.float32),
        ],
    )(xf, wg, W_kv, b_kv.reshape(1, 2 * HEAD_DIM))

    xb = xf.astype(jnp.bfloat16)
    wm_flat = W_map.transpose(1, 0, 2).reshape(DIM, NEXP * HEAD_DIM)
    wm_flat = wm_flat.astype(jnp.bfloat16)
    allq = pl.pallas_call(
        _allq_body,
        grid=(NT,),
        in_specs=[
            pl.BlockSpec((TILE, DIM), lambda i: (i, 0)),
            pl.BlockSpec((DIM, NEXP * HEAD_DIM), lambda i: (0, 0)),
        ],
        out_specs=pl.BlockSpec((TILE, NEXP * HEAD_DIM), lambda i: (i, 0)),
        out_shape=jax.ShapeDtypeStruct((NTOK, NEXP * HEAD_DIM), jnp.float32),
    )(xb, wm_flat)

    qidx_hm = qidx.T.reshape(1, NTOK * HEADS)
    qrows = _sc_gather_rows(allq.reshape(NTOK * NEXP // 2, 2 * HEAD_DIM),
                            qidx_hm)
    qhm = qrows.reshape(HEADS, NTOK, 2 * HEAD_DIM)

    k_ = kv[:, :HEAD_DIM].astype(jnp.bfloat16)
    v_ = kv[:, HEAD_DIM:].astype(jnp.bfloat16)
    y = pl.pallas_call(
        _attn_body,
        grid=(NT, HEADS),
        in_specs=[
            pl.BlockSpec((1, TILE, 2 * HEAD_DIM), lambda i, k: (k, i, 0)),
            pl.BlockSpec((NTOK, HEAD_DIM), lambda i, k: (0, 0)),
            pl.BlockSpec((NTOK, HEAD_DIM), lambda i, k: (0, 0)),
            pl.BlockSpec((TILE, HEADS), lambda i, k: (i, 0)),
            pl.BlockSpec((TILE, HEADS), lambda i, k: (i, 0)),
        ],
        out_specs=pl.BlockSpec((1, TILE, 2 * HEAD_DIM), lambda i, k: (k, i, 0)),
        out_shape=jax.ShapeDtypeStruct((HEADS, NTOK, 2 * HEAD_DIM),
                                       jnp.float32),
    )(qhm, k_, v_, g16, par16)

    scat_hm = scat.T.reshape(1, NTOK * HEADS)
    zrows = _sc_scatter_rows(y.reshape(HEADS * NTOK, 2 * HEAD_DIM),
                             scat_hm, NTOK * NEXP)
    z = zrows.reshape(NTOK, NEXP * 2 * HEAD_DIM)

    wo_pad = jnp.pad(W_out, ((0, 0), (0, HEAD_DIM), (0, 0)))
    wo_pad = wo_pad.reshape(NEXP * 2 * HEAD_DIM, DIM).astype(jnp.bfloat16)
    pexp = jnp.repeat(jnp.eye(NEXP, dtype=jnp.bfloat16), 2 * HEAD_DIM, axis=1)
    out = pl.pallas_call(
        _combine_body,
        grid=(NT,),
        in_specs=[
            pl.BlockSpec((TILE, NEXP * 2 * HEAD_DIM), lambda i: (i, 0)),
            pl.BlockSpec((TILE, NEXP), lambda i: (i, 0)),
            pl.BlockSpec((NEXP, NEXP * 2 * HEAD_DIM), lambda i: (0, 0)),
            pl.BlockSpec((NEXP * 2 * HEAD_DIM, DIM), lambda i: (0, 0)),
        ],
        out_specs=pl.BlockSpec((TILE, DIM), lambda i: (i, 0)),
        out_shape=jax.ShapeDtypeStruct((NTOK, DIM), jnp.float32),
    )(z, selmask, pexp, wo_pad)

    return out.reshape(Bb, Nn, C), aux[0, 0]
